# baseline (speedup 1.0000x reference)
import functools

import jax
import jax.numpy as jnp
from jax import lax
from jax.experimental import pallas as pl
from jax.experimental.pallas import tpu as pltpu
from jax.experimental.pallas import tpu_sc as plsc

NUM_EMBEDDINGS = 1000000
EMBEDDING_DIM = 64

NBATCH = 4096
NSEQ = 50
NUM_CORES = 2
NUM_SUBCORES = 16
NW = NUM_CORES * NUM_SUBCORES
COLS_PER_W = NBATCH // NW
NBUF = 5
NGROUPS = NSEQ // NBUF


def _gather_body(idx_hbm, table_hbm, out_hbm, idx_v, buf_v, *sems):
    gsems = sems[:NBUF]
    wsems = sems[NBUF:]
    wid = lax.axis_index("s") * NUM_CORES + lax.axis_index("c")
    col0 = wid * COLS_PER_W
    pltpu.sync_copy(idx_hbm.at[:, pl.ds(col0, COLS_PER_W)], idx_v)

    def gather_start(s, b):
        pltpu.async_copy(table_hbm.at[idx_v.at[s]], buf_v.at[b], gsems[b])

    def gather_wait(b):
        pltpu.make_async_copy(
            table_hbm.at[idx_v.at[0]], buf_v.at[b], gsems[b]
        ).wait()

    def write_start(s, b):
        pltpu.async_copy(
            buf_v.at[b], out_hbm.at[pl.ds(col0, COLS_PER_W), s], wsems[b]
        )

    def write_wait(b):
        pltpu.make_async_copy(
            buf_v.at[b], out_hbm.at[pl.ds(col0, COLS_PER_W), 0], wsems[b]
        ).wait()

    for b in range(NBUF):
        gather_start(b, b)

    def group(g, carry):
        for b in range(NBUF):
            s = g * NBUF + b
            gather_wait(b)
            write_start(s, b)

            @pl.when(g < NGROUPS - 1)
            def _():
                write_wait(b)
                gather_start(s + NBUF, b)

        return carry

    lax.fori_loop(0, NGROUPS, group, 0)
    for b in range(NBUF):
        write_wait(b)


def kernel(input, weight):
    idx_t = input.T
    mesh = plsc.VectorSubcoreMesh(core_axis_name="c", subcore_axis_name="s")
    run = functools.partial(
        pl.kernel,
        mesh=mesh,
        out_type=jax.ShapeDtypeStruct((NBATCH, NSEQ, EMBEDDING_DIM), jnp.float32),
        scratch_types=[
            pltpu.VMEM((NSEQ, COLS_PER_W), jnp.int32),
            pltpu.VMEM((NBUF, COLS_PER_W, EMBEDDING_DIM), jnp.float32),
        ]
        + [pltpu.SemaphoreType.DMA] * (2 * NBUF),
        compiler_params=pltpu.CompilerParams(use_tc_tiling_on_sc=False),
    )(_gather_body)
    return run(idx_t, weight)
